# baseline (speedup 1.0000x reference)
import functools

import jax
import jax.numpy as jnp
from jax import lax
from jax.experimental import pallas as pl
from jax.experimental.pallas import tpu as pltpu
from jax.experimental.pallas import tpu_sc as plsc

VOCAB = 13
EMBED_DIM = 1024
BATCH = 4
SEQ = 8192

_B = BATCH * SEQ
_NW = 32
_BPW = _B // _NW
_C = 64
_NCH = _BPW // _C

_mesh = plsc.VectorSubcoreMesh(core_axis_name="c", subcore_axis_name="s")


@functools.partial(
    pl.kernel,
    mesh=_mesh,
    out_type=jax.ShapeDtypeStruct((_B, EMBED_DIM), jnp.float32),
    scratch_types=[
        pltpu.VMEM((_BPW,), jnp.int32),
        pltpu.VMEM((_C, EMBED_DIM), jnp.float32),
        pltpu.SemaphoreType.DMA,
    ],
)
def _emb(x_hbm, table_hbm, out_hbm, idx_v, rows_v, sem):
    wid = lax.axis_index("s") * 2 + lax.axis_index("c")
    base = wid * _BPW
    pltpu.sync_copy(x_hbm.at[pl.ds(base, _BPW)], idx_v)

    def body(j, carry):
        cbase = j * _C
        pltpu.async_copy(
            table_hbm.at[idx_v.at[pl.ds(cbase, _C)]], rows_v, sem
        ).wait()
        pltpu.sync_copy(rows_v, out_hbm.at[pl.ds(base + cbase, _C)])
        return carry

    lax.fori_loop(0, _NCH, body, 0)


def kernel(x, table):
    out = _emb(x.reshape(_B).astype(jnp.int32), table)
    return out.reshape(BATCH, SEQ, EMBED_DIM)


# speedup vs baseline: 1.3715x; 1.3715x over previous
import functools

import jax
import jax.numpy as jnp
from jax import lax
from jax.experimental import pallas as pl
from jax.experimental.pallas import tpu as pltpu
from jax.experimental.pallas import tpu_sc as plsc

VOCAB = 13
EMBED_DIM = 1024
BATCH = 4
SEQ = 8192

_B = BATCH * SEQ
_NW = 32
_BPW = _B // _NW
_C = 32
_NCH = _BPW // _C
_VPAD = 16

_mesh = plsc.VectorSubcoreMesh(core_axis_name="c", subcore_axis_name="s")


@functools.partial(
    pl.kernel,
    mesh=_mesh,
    out_type=jax.ShapeDtypeStruct((_B, EMBED_DIM), jnp.float32),
    scratch_types=[
        pltpu.VMEM((_BPW,), jnp.int32),
        pltpu.VMEM((_C, EMBED_DIM), jnp.float32),
        pltpu.VMEM((_C, EMBED_DIM), jnp.float32),
        pltpu.SemaphoreType.DMA,
        pltpu.SemaphoreType.DMA,
        pltpu.SemaphoreType.DMA,
        pltpu.SemaphoreType.DMA,
    ],
)
def _emb(x_hbm, table_hbm, out_hbm, idx_v, rows0, rows1,
         gsem0, gsem1, ssem0, ssem1):
    sid = lax.axis_index("s")
    wid = sid * 2 + lax.axis_index("c")
    base = wid * _BPW

    pltpu.sync_copy(x_hbm.at[pl.ds(base, _BPW)], idx_v)

    rows = (rows0, rows1)
    gsem = (gsem0, gsem1)
    ssem = (ssem0, ssem1)

    def g_start(b, j):
        pltpu.async_copy(table_hbm.at[idx_v.at[pl.ds(j * _C, _C)]],
                         rows[b], gsem[b])

    def g_wait(b, j):
        pltpu.make_async_copy(table_hbm.at[idx_v.at[pl.ds(j * _C, _C)]],
                              rows[b], gsem[b]).wait()

    def s_start(b, j):
        pltpu.async_copy(rows[b], out_hbm.at[pl.ds(base + j * _C, _C)],
                         ssem[b])

    def s_wait(b, j):
        pltpu.make_async_copy(rows[b], out_hbm.at[pl.ds(base + j * _C, _C)],
                              ssem[b]).wait()

    g_start(0, 0)
    g_wait(0, 0)
    s_start(0, 0)
    g_start(1, 1)

    def body(jj, carry):
        for b, j in ((1, 2 * jj + 1), (0, 2 * jj + 2)):
            g_wait(b, j)
            s_start(b, j)
            s_wait(1 - b, j - 1)
            g_start(1 - b, j + 1)
        return carry

    lax.fori_loop(0, (_NCH - 2) // 2, body, 0)

    g_wait(1, _NCH - 1)
    s_start(1, _NCH - 1)
    s_wait(0, _NCH - 2)
    s_wait(1, _NCH - 1)


def kernel(x, table):
    table_padded = jnp.pad(table, ((0, _VPAD - VOCAB), (0, 0)))
    out = _emb(x.reshape(_B).astype(jnp.int32), table_padded)
    return out.reshape(BATCH, SEQ, EMBED_DIM)
